# baseline (speedup 1.0000x reference)
import functools

import jax
import jax.numpy as jnp
from jax import lax
from jax.experimental import pallas as pl
from jax.experimental.pallas import tpu as pltpu
from jax.experimental.pallas import tpu_sc as plsc

_D = 64
_B = 16384
_NC = 2
_NS = 16
_NW = _NC * _NS
_BPW = _B // _NW
_L = 16


def _sc_body(idx_hbm, table_hbm, w_hbm, out_hbm, idx_v, rows_v, spm, w_v,
             out_v, sem, sem2):
    wid = lax.axis_index("s") * _NC + lax.axis_index("c")
    sid = lax.axis_index("s")
    base = wid * _BPW

    pltpu.sync_copy(idx_hbm.at[pl.ds(base, _BPW)], idx_v)
    pltpu.sync_copy(w_hbm, w_v)

    half = _BPW // 2

    def issue(j, carry):
        ivec = plsc.load_gather(idx_v, [jnp.full((_L,), j)])
        i = jnp.max(ivec)
        pltpu.async_copy(table_hbm.at[i], rows_v.at[j], sem)
        ivec2 = plsc.load_gather(idx_v, [jnp.full((_L,), half + j)])
        i2 = jnp.max(ivec2)
        pltpu.async_copy(table_hbm.at[i2], spm.at[sid, j], sem2)
        return carry

    lax.fori_loop(0, half, issue, 0)
    pltpu.make_async_copy(
        table_hbm.at[pl.ds(0, half)], rows_v.at[pl.ds(0, half)], sem
    ).wait()
    pltpu.make_async_copy(
        table_hbm.at[pl.ds(0, half)], spm.at[sid], sem2
    ).wait()
    pltpu.sync_copy(spm.at[sid], rows_v.at[pl.ds(half, half)])

    iota = lax.iota(jnp.int32, _L)
    wqs = [w_v[pl.ds(k * _L, _L)] for k in range(_D // _L)]
    tail = iota == (_L - 1)

    def body(j, carry):
        acc = jnp.zeros((_L,), jnp.float32)
        for k in range(_D // _L):
            acc = acc + rows_v[j, pl.ds(k * _L, _L)] * wqs[k]
        tot = plsc.cumsum(acc)
        plsc.store_scatter(out_v, [jnp.full((_L,), j)], tot, mask=tail)
        return carry

    lax.fori_loop(0, _BPW, body, 0)

    pltpu.sync_copy(out_v, out_hbm.at[pl.ds(base, _BPW)])


_gather_reduce = functools.partial(
    pl.kernel,
    mesh=plsc.VectorSubcoreMesh(core_axis_name="c", subcore_axis_name="s"),
    out_type=jax.ShapeDtypeStruct((_B,), jnp.float32),
    compiler_params=pltpu.CompilerParams(needs_layout_passes=False),
    scratch_types=[
        pltpu.VMEM((_BPW,), jnp.int32),
        pltpu.VMEM((_BPW, _D), jnp.float32),
        pltpu.VMEM_SHARED((_NS, _BPW // 2, _D), jnp.float32),
        pltpu.VMEM((2 * _D,), jnp.float32),
        pltpu.VMEM((_BPW,), jnp.float32),
        pltpu.SemaphoreType.DMA,
        pltpu.SemaphoreType.DMA,
    ],
)(_sc_body)


@jax.jit
def kernel(indices, table, W):
    w = jnp.pad(W.reshape(_D), (0, _D))
    out = _gather_reduce(indices, table, w)
    return out.reshape(_B, 1)


# speedup vs baseline: 1.0219x; 1.0219x over previous
import functools

import jax
import jax.numpy as jnp
from jax import lax
from jax.experimental import pallas as pl
from jax.experimental.pallas import tpu as pltpu
from jax.experimental.pallas import tpu_sc as plsc

_D = 64
_B = 16384
_NC = 2
_NS = 16
_NW = _NC * _NS
_BPW = _B // _NW
_L = 16


def _sc_body(idx_hbm, table_hbm, w_hbm, out_hbm, idx_v, rows_v, w_v, out_v,
             sem):
    wid = lax.axis_index("s") * _NC + lax.axis_index("c")
    base = wid * _BPW

    pltpu.sync_copy(idx_hbm.at[pl.ds(base, _BPW)], idx_v)
    pltpu.sync_copy(w_hbm, w_v)

    def issue(j, carry):
        ivec = plsc.load_gather(idx_v, [jnp.full((_L,), j)])
        i = jnp.max(ivec)
        pltpu.async_copy(table_hbm.at[i], rows_v.at[j], sem)
        return carry

    lax.fori_loop(0, _BPW, issue, 0)
    pltpu.make_async_copy(table_hbm.at[pl.ds(0, _BPW)], rows_v, sem).wait()

    iota = lax.iota(jnp.int32, _L)
    wqs = [w_v[pl.ds(k * _L, _L)] for k in range(_D // _L)]
    tail = iota == (_L - 1)

    def body(j, carry):
        acc = jnp.zeros((_L,), jnp.float32)
        for k in range(_D // _L):
            acc = acc + rows_v[j, pl.ds(k * _L, _L)] * wqs[k]
        tot = plsc.cumsum(acc)
        plsc.store_scatter(out_v, [jnp.full((_L,), j)], tot, mask=tail)
        return carry

    lax.fori_loop(0, _BPW, body, 0)

    pltpu.sync_copy(out_v, out_hbm.at[pl.ds(base, _BPW)])


_gather_reduce = functools.partial(
    pl.kernel,
    mesh=plsc.VectorSubcoreMesh(core_axis_name="c", subcore_axis_name="s"),
    out_type=jax.ShapeDtypeStruct((_B,), jnp.float32),
    compiler_params=pltpu.CompilerParams(needs_layout_passes=False),
    scratch_types=[
        pltpu.VMEM((_BPW,), jnp.int32),
        pltpu.VMEM((_BPW, _D), jnp.float32),
        pltpu.VMEM((2 * _D,), jnp.float32),
        pltpu.VMEM((_BPW,), jnp.float32),
        pltpu.SemaphoreType.DMA,
    ],
)(_sc_body)


@jax.jit
def kernel(indices, table, W):
    w = jnp.pad(W.reshape(_D), (0, _D))
    out = _gather_reduce(indices, table, w)
    return out.reshape(_B, 1)
